# Initial kernel scaffold; baseline (speedup 1.0000x reference)
import jax
import jax.numpy as jnp
from jax.experimental import pallas as pl


def kernel(image):
    raise NotImplementedError("write your pallas kernel here")


# speedup vs baseline: 99.5718x; 99.5718x over previous
import jax
import jax.numpy as jnp
from jax.experimental import pallas as pl
from jax.experimental.pallas import tpu as pltpu

H, W = 2048, 4096
NLAB = 3
NCH = 3 + NLAB
HH = H // 2

BR_A = 128
NJ_A = HH // BR_A
BR_C = 128
NJ_C = HH // BR_C


def _hist_kernel(img_ref, hist_ref):
    j = pl.program_id(2)
    iota16 = jax.lax.broadcasted_iota(jnp.int32, (16, 1, 1), 0).astype(jnp.bfloat16)
    one = jnp.ones((), jnp.bfloat16)
    zero = jnp.zeros((), jnp.bfloat16)

    def chunk(g, acc):
        v = img_ref[0, pl.ds(g * 8, 8), :]
        vf = jnp.clip(jnp.floor(v), 0.0, 255.0)
        hi = jnp.floor(vf * (1.0 / 16.0))
        lo = vf - hi * 16.0
        hi_bf = hi.astype(jnp.bfloat16)
        lo_bf = lo.astype(jnp.bfloat16)
        ohh = jnp.where(hi_bf[None, :, :] == iota16, one, zero)
        ohl = jnp.where(lo_bf[None, :, :] == iota16, one, zero)
        for r in range(8):
            acc = acc + jax.lax.dot_general(
                ohh[:, r, :], ohl[:, r, :],
                (((1,), (1,)), ((), ())),
                preferred_element_type=jnp.float32)
        return acc

    acc = jax.lax.fori_loop(0, BR_A // 8, chunk,
                            jnp.zeros((16, 16), jnp.float32))

    @pl.when(j == 0)
    def _():
        hist_ref[0, 0] = acc

    @pl.when(j > 0)
    def _():
        hist_ref[0, 0] = hist_ref[0, 0] + acc


def _fdiv(a, d):
    q = jnp.floor(a / d)
    r = a - q * d
    for _ in range(3):
        over = r >= d
        q = q + jnp.where(over, 1.0, 0.0)
        r = r - jnp.where(over, d, 0.0)
        under = r < 0.0
        q = q - jnp.where(under, 1.0, 0.0)
        r = r + jnp.where(under, d, 0.0)
    return q


def _shift_lanes_right(x, k):
    return jnp.pad(x, ((0, 0), (k, 0)))[:, :16]


def _shift_rows_down(x, k):
    return jnp.pad(x, ((k, 0), (0, 0)))[:16, :]


def _lut_kernel(hist_ref, lut_ref):
    iota_i = jax.lax.broadcasted_iota(jnp.int32, (16, 16), 0).astype(jnp.float32)
    iota_j = jax.lax.broadcasted_iota(jnp.int32, (16, 16), 1).astype(jnp.float32)
    idxmat = iota_i * 16.0 + iota_j

    for c in range(3):
        h = hist_ref[0, c] + hist_ref[1, c]
        rc = h
        for k in (1, 2, 4, 8):
            rc = rc + _shift_lanes_right(rc, k)
        rowtot = rc[:, 15:16]
        pr = rowtot
        for k in (1, 2, 4, 8):
            pr = pr + _shift_rows_down(pr, k)
        cum = rc + (pr - rowtot)
        total = jnp.sum(h, keepdims=True)
        masked = jnp.where(h > 0.0, idxmat, -1.0)
        last_nz = jnp.max(masked, keepdims=True)
        h_last = jnp.sum(jnp.where(idxmat == last_nz, h, 0.0), keepdims=True)
        step = _fdiv(total - h_last, jnp.full((1, 1), 255.0))
        d = jnp.maximum(step, 1.0)
        a = cum + jnp.floor(step * 0.5)
        q = _fdiv(a, d)
        qs = _shift_lanes_right(q, 1)
        prevrowlast = _shift_rows_down(q[:, 15:16], 1)
        lut = qs + jnp.where(iota_j == 0.0, prevrowlast, 0.0)
        lut = jnp.clip(lut, 0.0, 255.0)
        lut = jnp.where(step == 0.0, idxmat, lut)
        lut_ref[c] = lut


def _remap_kernel(img_ref, lut_ref, out_ref):
    c = pl.program_id(1)

    @pl.when(c < 3)
    def _():
        v = img_ref[0]
        idx = jnp.clip(v, 0.0, 255.0).astype(jnp.int32)
        idxm = jnp.bitwise_and(idx, 127)
        lo_rows = jnp.broadcast_to(lut_ref[0, 0:1, :], (BR_C, 128))
        hi_rows = jnp.broadcast_to(lut_ref[0, 1:2, :], (BR_C, 128))
        g_lo = jnp.take_along_axis(lo_rows, idxm, axis=1)
        g_hi = jnp.take_along_axis(hi_rows, idxm, axis=1)
        out_ref[0] = jnp.where(idx >= 128, g_hi, g_lo)

    @pl.when(c >= 3)
    def _():
        out_ref[0] = img_ref[0]


def kernel(image):
    part = pl.pallas_call(
        _hist_kernel,
        grid=(2, 3, NJ_A),
        in_specs=[pl.BlockSpec((1, BR_A, W),
                               lambda p, c, j: (c, p * NJ_A + j, 0))],
        out_specs=pl.BlockSpec((1, 1, 16, 16), lambda p, c, j: (p, c, 0, 0)),
        out_shape=jax.ShapeDtypeStruct((2, 3, 16, 16), jnp.float32),
        compiler_params=pltpu.CompilerParams(
            dimension_semantics=("parallel", "arbitrary", "arbitrary")),
    )(image)

    lut3d = pl.pallas_call(
        _lut_kernel,
        grid=(1,),
        in_specs=[pl.BlockSpec((2, 3, 16, 16), lambda i: (0, 0, 0, 0))],
        out_specs=pl.BlockSpec((3, 16, 16), lambda i: (0, 0, 0)),
        out_shape=jax.ShapeDtypeStruct((3, 16, 16), jnp.float32),
        compiler_params=pltpu.CompilerParams(
            dimension_semantics=("arbitrary",)),
    )(part)

    lut = lut3d.reshape(3, 2, 128)

    out = pl.pallas_call(
        _remap_kernel,
        grid=(2, NCH, NJ_C),
        in_specs=[
            pl.BlockSpec((1, BR_C, W), lambda p, c, j: (c, p * NJ_C + j, 0)),
            pl.BlockSpec((1, 2, 128),
                         lambda p, c, j: (jnp.minimum(c, 2), 0, 0)),
        ],
        out_specs=pl.BlockSpec((1, BR_C, W),
                               lambda p, c, j: (c, p * NJ_C + j, 0)),
        out_shape=jax.ShapeDtypeStruct((NCH, H, W), jnp.float32),
        compiler_params=pltpu.CompilerParams(
            dimension_semantics=("parallel", "arbitrary", "arbitrary")),
    )(image, lut)

    return out
